# Initial kernel scaffold; baseline (speedup 1.0000x reference)
import jax
import jax.numpy as jnp
from jax.experimental import pallas as pl


def kernel(seq1, seq2, adj_idx, adj_val, aug1_idx, aug1_val, aug2_idx, aug2_val, W, b_gcn, prelu_a, Wb, bb):
    raise NotImplementedError("write your pallas kernel here")


# speedup vs baseline: 4.6844x; 4.6844x over previous
import jax
import jax.numpy as jnp
from jax import lax
from jax.experimental import pallas as pl
from jax.experimental.pallas import tpu as pltpu
from jax.experimental.pallas import tpu_sc as plsc

_N = 10000
_E = 320000
_D = 128
_NT = 16
_KB = 100
_EPT = _E // _NT
_NBATCH = _EPT // _KB
_ACC_N = 10240
_RPT = _ACC_N // _NT
_CB = 40
_ZR = 80


def _mm_body(x1_ref, x2_ref, w_ref, o1_ref, o2_ref):
    w = w_ref[...]
    o1_ref[...] = jnp.dot(x1_ref[...], w, preferred_element_type=jnp.float32)
    o2_ref[...] = jnp.dot(x2_ref[...], w, preferred_element_type=jnp.float32)


def _features(x1, x2, W):
    return pl.pallas_call(
        _mm_body,
        out_shape=(
            jax.ShapeDtypeStruct((_N, _D), jnp.float32),
            jax.ShapeDtypeStruct((_N, _D), jnp.float32),
        ),
    )(x1, x2, W)


def _one_spmm(dst2, src2, val1, fts, out, acc,
              src_t, dst_t, val_t, rows, sem, tid):
    def zb(i, c):
        for d in range(8):
            rows[i, pl.ds(d * 16, 16)] = jnp.zeros((16,), jnp.float32)
        return c

    lax.fori_loop(0, _ZR, zb, 0, unroll=False)
    zrow = tid * _RPT
    for i in range(_RPT // _ZR):
        pltpu.sync_copy(rows.at[pl.ds(0, _ZR)], acc.at[pl.ds(zrow + i * _ZR, _ZR)])
    plsc.subcore_barrier()

    def chunk(ci, carry):
        base = tid * _NBATCH + ci * _CB
        pltpu.sync_copy(src2.at[pl.ds(base, _CB)], src_t)
        pltpu.sync_copy(dst2.at[pl.ds(base, _CB)], dst_t)
        pltpu.sync_copy(val1.at[pl.ds(tid * _EPT + ci * _CB * _KB, _CB * _KB)],
                        val_t)

        def batch(bi, c1):
            pltpu.async_copy(fts.at[src_t.at[bi]], rows, sem).wait()

            def scale(e, c2):
                vv = plsc.load_gather(
                    val_t, [jnp.full((16,), bi * _KB, jnp.int32) + e]
                )
                for d in range(8):
                    sl = pl.ds(d * 16, 16)
                    rows[e, sl] = rows[e, sl] * vv
                return c2

            lax.fori_loop(0, _KB, scale, 0, unroll=False)
            pltpu.sync_copy(rows, acc.at[dst_t.at[bi]], add=True)
            return c1

        lax.fori_loop(0, _CB, batch, 0, unroll=False)
        return carry

    lax.fori_loop(0, _NBATCH // _CB, chunk, 0, unroll=False)
    plsc.subcore_barrier()

    orow = tid * _RPT

    @pl.when(tid < _NT - 1)
    def _():
        pltpu.sync_copy(acc.at[pl.ds(orow, _RPT)], out.at[pl.ds(orow, _RPT)])

    @pl.when(tid == _NT - 1)
    def _():
        lo = (_NT - 1) * _RPT
        pltpu.sync_copy(acc.at[pl.ds(lo, _N - lo)], out.at[pl.ds(lo, _N - lo)])

    plsc.subcore_barrier()


def _spmm_body(fts1, fts2,
               dA, sA, vA, d1, s1, v1, d2, s2, v2,
               s0_out, s1_out, s2_out, s3_out,
               acc, src_t, dst_t, val_t, rows, sem):
    cid = lax.axis_index("c")
    tid = lax.axis_index("s")

    @pl.when(cid == 0)
    def _():
        _one_spmm(dA, sA, vA, fts1, s0_out, acc,
                  src_t, dst_t, val_t, rows, sem, tid)
        _one_spmm(d1, s1, v1, fts1, s1_out, acc,
                  src_t, dst_t, val_t, rows, sem, tid)

    @pl.when(cid == 1)
    def _():
        _one_spmm(d2, s2, v2, fts1, s2_out, acc,
                  src_t, dst_t, val_t, rows, sem, tid)
        _one_spmm(dA, sA, vA, fts2, s3_out, acc,
                  src_t, dst_t, val_t, rows, sem, tid)


def _spmm_all(fts1, fts2, dA, sA, vA, d1, s1, v1, d2, s2, v2):
    f32 = jnp.float32
    run = pl.kernel(
        _spmm_body,
        out_type=[jax.ShapeDtypeStruct((_N, _D), f32) for _ in range(4)],
        mesh=plsc.VectorSubcoreMesh(core_axis_name="c", subcore_axis_name="s"),
        compiler_params=pltpu.CompilerParams(needs_layout_passes=False),
        scratch_types=[
            pltpu.VMEM_SHARED((_ACC_N, _D), f32),
            pltpu.VMEM((_CB, _KB), jnp.int32),
            pltpu.VMEM((_CB, _KB), jnp.int32),
            pltpu.VMEM((_CB * _KB,), f32),
            pltpu.VMEM((_KB, _D), f32),
            pltpu.SemaphoreType.DMA,
        ],
    )
    return run(fts1, fts2, dA, sA, vA, d1, s1, v1, d2, s2, v2)


def _fin_body(s0_ref, s1_ref, s2_ref, s3_ref, b_ref, a_ref, wb_ref, bb_ref,
              out_ref):
    a = a_ref[0, 0]
    b = b_ref[...]

    def prelu(x):
        return jnp.where(x > 0, x, a * x)

    h1 = prelu(s1_ref[...] + b)
    h3 = prelu(s2_ref[...] + b)
    c = jax.nn.sigmoid(jnp.mean(h1, axis=0)) + jax.nn.sigmoid(jnp.mean(h3, axis=0))
    v = jnp.sum(wb_ref[...] * c[None, :], axis=1)

    h0 = prelu(s0_ref[...] + b)
    h2 = prelu(s3_ref[...] + b)
    two_bb = 2.0 * bb_ref[0, 0]
    out_ref[0, :] = jnp.sum(h0 * v[None, :], axis=1) + two_bb
    out_ref[1, :] = jnp.sum(h2 * v[None, :], axis=1) + two_bb


def _finalize(s0, s1, s2, s3, b, a, Wb, bb):
    return pl.pallas_call(
        _fin_body,
        out_shape=jax.ShapeDtypeStruct((2, _N), jnp.float32),
    )(s0, s1, s2, s3, b, a, Wb, bb)


def kernel(seq1, seq2, adj_idx, adj_val, aug1_idx, aug1_val, aug2_idx,
           aug2_val, W, b_gcn, prelu_a, Wb, bb):
    fts1, fts2 = _features(seq1[0], seq2[0], W)

    def layout(idx, val):
        dst = idx[0].reshape(_E // _KB, _KB)
        src = idx[1].reshape(_E // _KB, _KB)
        return dst, src, val

    dA, sA, vA = layout(adj_idx, adj_val)
    d1, s1, v1 = layout(aug1_idx, aug1_val)
    d2, s2, v2 = layout(aug2_idx, aug2_val)

    s0, s1_, s2_, s3 = _spmm_all(fts1, fts2, dA, sA, vA, d1, s1, v1, d2, s2, v2)

    out2 = _finalize(s0, s1_, s2_, s3,
                     b_gcn.reshape(1, _D),
                     prelu_a.reshape(1, 1),
                     Wb,
                     bb.reshape(1, 1))
    return out2.reshape(1, 2 * _N)


# speedup vs baseline: 8.3873x; 1.7905x over previous
import jax
import jax.numpy as jnp
from jax import lax
from jax.experimental import pallas as pl
from jax.experimental.pallas import tpu as pltpu
from jax.experimental.pallas import tpu_sc as plsc

_N = 10000
_E = 320000
_D = 128
_NT = 16
_KB = 100
_EPT = _E // _NT
_NBATCH = _EPT // _KB
_ACC_N = 10240
_RPT = _ACC_N // _NT
_CB = 40
_ZR = 80


def _mm_body(x1_ref, x2_ref, w_ref, o1_ref, o2_ref):
    w = w_ref[...]
    o1_ref[...] = jnp.dot(x1_ref[...], w, preferred_element_type=jnp.float32)
    o2_ref[...] = jnp.dot(x2_ref[...], w, preferred_element_type=jnp.float32)


def _features(x1, x2, W):
    return pl.pallas_call(
        _mm_body,
        out_shape=(
            jax.ShapeDtypeStruct((_N, _D), jnp.float32),
            jax.ShapeDtypeStruct((_N, _D), jnp.float32),
        ),
    )(x1, x2, W)


def _scale_rows(rows, val_t, boff):
    @plsc.parallel_loop(0, _KB, step=1, unroll=4)
    def _(e):
        vv = plsc.load_gather(val_t, [jnp.full((16,), boff, jnp.int32) + e])
        for d in range(8):
            sl = pl.ds(d * 16, 16)
            rows[e, sl] = rows[e, sl] * vv


def _one_spmm(dst2, src2, val1, fts, out, acc,
              src_t, dst_t, val_t, rows0, rows1,
              gsem0, gsem1, ssem0, ssem1, tid):
    def zb(i, c):
        for d in range(8):
            rows0[i, pl.ds(d * 16, 16)] = jnp.zeros((16,), jnp.float32)
        return c

    lax.fori_loop(0, _ZR, zb, 0, unroll=False)
    zrow = tid * _RPT
    for i in range(_RPT // _ZR):
        pltpu.sync_copy(rows0.at[pl.ds(0, _ZR)], acc.at[pl.ds(zrow + i * _ZR, _ZR)])
    plsc.subcore_barrier()

    n_pairs = _CB // 2

    def chunk(ci, carry):
        base = tid * _NBATCH + ci * _CB
        pltpu.sync_copy(src2.at[pl.ds(base, _CB)], src_t)
        pltpu.sync_copy(dst2.at[pl.ds(base, _CB)], dst_t)
        pltpu.sync_copy(val1.at[pl.ds(tid * _EPT + ci * _CB * _KB, _CB * _KB)],
                        val_t)
        pltpu.async_copy(fts.at[src_t.at[0]], rows0, gsem0)

        def pair(si, c1):
            b0 = 2 * si
            b1 = b0 + 1
            @pl.when(si > 0)
            def _():
                pltpu.make_async_copy(rows1, acc.at[dst_t.at[b1]], ssem1).wait()

            d_g1 = pltpu.async_copy(fts.at[src_t.at[b1]], rows1, gsem1)
            pltpu.make_async_copy(fts.at[src_t.at[b0]], rows0, gsem0).wait()
            _scale_rows(rows0, val_t, b0 * _KB)
            d_s0 = pltpu.async_copy(rows0, acc.at[dst_t.at[b0]], ssem0, add=True)
            d_g1.wait()
            _scale_rows(rows1, val_t, b1 * _KB)
            pltpu.async_copy(rows1, acc.at[dst_t.at[b1]], ssem1, add=True)
            d_s0.wait()

            @pl.when(si < n_pairs - 1)
            def _():
                pltpu.async_copy(fts.at[src_t.at[b0 + 2]], rows0, gsem0)

            return c1

        lax.fori_loop(0, n_pairs, pair, 0, unroll=False)
        pltpu.make_async_copy(rows1, acc.at[dst_t.at[0]], ssem1).wait()
        return carry

    lax.fori_loop(0, _NBATCH // _CB, chunk, 0, unroll=False)
    plsc.subcore_barrier()

    orow = tid * _RPT

    @pl.when(tid < _NT - 1)
    def _():
        pltpu.sync_copy(acc.at[pl.ds(orow, _RPT)], out.at[pl.ds(orow, _RPT)])

    @pl.when(tid == _NT - 1)
    def _():
        lo = (_NT - 1) * _RPT
        pltpu.sync_copy(acc.at[pl.ds(lo, _N - lo)], out.at[pl.ds(lo, _N - lo)])

    plsc.subcore_barrier()


def _spmm_body(fts1, fts2,
               dA, sA, vA, d1, s1, v1, d2, s2, v2,
               s0_out, s1_out, s2_out, s3_out,
               acc, src_t, dst_t, val_t, rows0, rows1,
               gsem0, gsem1, ssem0, ssem1):
    cid = lax.axis_index("c")
    tid = lax.axis_index("s")

    args = (src_t, dst_t, val_t, rows0, rows1, gsem0, gsem1, ssem0, ssem1, tid)

    @pl.when(cid == 0)
    def _():
        _one_spmm(dA, sA, vA, fts1, s0_out, acc, *args)
        _one_spmm(d1, s1, v1, fts1, s1_out, acc, *args)

    @pl.when(cid == 1)
    def _():
        _one_spmm(d2, s2, v2, fts1, s2_out, acc, *args)
        _one_spmm(dA, sA, vA, fts2, s3_out, acc, *args)


def _spmm_all(fts1, fts2, dA, sA, vA, d1, s1, v1, d2, s2, v2):
    f32 = jnp.float32
    run = pl.kernel(
        _spmm_body,
        out_type=[jax.ShapeDtypeStruct((_N, _D), f32) for _ in range(4)],
        mesh=plsc.VectorSubcoreMesh(core_axis_name="c", subcore_axis_name="s"),
        compiler_params=pltpu.CompilerParams(needs_layout_passes=False),
        scratch_types=[
            pltpu.VMEM_SHARED((_ACC_N, _D), f32),
            pltpu.VMEM((_CB, _KB), jnp.int32),
            pltpu.VMEM((_CB, _KB), jnp.int32),
            pltpu.VMEM((_CB * _KB,), f32),
            pltpu.VMEM((_KB, _D), f32),
            pltpu.VMEM((_KB, _D), f32),
            pltpu.SemaphoreType.DMA,
            pltpu.SemaphoreType.DMA,
            pltpu.SemaphoreType.DMA,
            pltpu.SemaphoreType.DMA,
        ],
    )
    return run(fts1, fts2, dA, sA, vA, d1, s1, v1, d2, s2, v2)


def _fin_body(s0_ref, s1_ref, s2_ref, s3_ref, b_ref, a_ref, wb_ref, bb_ref,
              out_ref):
    a = a_ref[0, 0]
    b = b_ref[...]

    def prelu(x):
        return jnp.where(x > 0, x, a * x)

    h1 = prelu(s1_ref[...] + b)
    h3 = prelu(s2_ref[...] + b)
    c = jax.nn.sigmoid(jnp.mean(h1, axis=0)) + jax.nn.sigmoid(jnp.mean(h3, axis=0))
    v = jnp.sum(wb_ref[...] * c[None, :], axis=1)

    h0 = prelu(s0_ref[...] + b)
    h2 = prelu(s3_ref[...] + b)
    two_bb = 2.0 * bb_ref[0, 0]
    out_ref[0, :] = jnp.sum(h0 * v[None, :], axis=1) + two_bb
    out_ref[1, :] = jnp.sum(h2 * v[None, :], axis=1) + two_bb


def _finalize(s0, s1, s2, s3, b, a, Wb, bb):
    return pl.pallas_call(
        _fin_body,
        out_shape=jax.ShapeDtypeStruct((2, _N), jnp.float32),
    )(s0, s1, s2, s3, b, a, Wb, bb)


def kernel(seq1, seq2, adj_idx, adj_val, aug1_idx, aug1_val, aug2_idx,
           aug2_val, W, b_gcn, prelu_a, Wb, bb):
    fts1, fts2 = _features(seq1[0], seq2[0], W)

    def layout(idx, val):
        dst = idx[0].reshape(_E // _KB, _KB)
        src = idx[1].reshape(_E // _KB, _KB)
        return dst, src, val

    dA, sA, vA = layout(adj_idx, adj_val)
    d1, s1, v1 = layout(aug1_idx, aug1_val)
    d2, s2, v2 = layout(aug2_idx, aug2_val)

    s0, s1_, s2_, s3 = _spmm_all(fts1, fts2, dA, sA, vA, d1, s1, v1, d2, s2, v2)

    out2 = _finalize(s0, s1_, s2_, s3,
                     b_gcn.reshape(1, _D),
                     prelu_a.reshape(1, 1),
                     Wb,
                     bb.reshape(1, 1))
    return out2.reshape(1, 2 * _N)


# speedup vs baseline: 9.1517x; 1.0911x over previous
import jax
import jax.numpy as jnp
from jax import lax
from jax.experimental import pallas as pl
from jax.experimental.pallas import tpu as pltpu
from jax.experimental.pallas import tpu_sc as plsc

_N = 10000
_E = 320000
_D = 128
_DW = _D // 2
_NT = 16
_KB = 50
_EPT = _E // _NT
_NBATCH = _EPT // _KB
_ACC_N = 10240
_RPT = _ACC_N // _NT
_CB = 40
_NCHUNK = _NBATCH // _CB


def _mm_body(x1_ref, x2_ref, w_ref, o1_ref, o2_ref):
    w = w_ref[...]

    def pack(x):
        o = jnp.dot(x, w, preferred_element_type=jnp.float32)
        u = lax.bitcast_convert_type(o.astype(jnp.bfloat16), jnp.uint16)
        lo = u[:, :_DW].astype(jnp.uint32)
        hi = u[:, _DW:].astype(jnp.uint32)
        return lo | (hi << 16)

    o1_ref[...] = pack(x1_ref[...])
    o2_ref[...] = pack(x2_ref[...])


def _features(x1, x2, W):
    return pl.pallas_call(
        _mm_body,
        out_shape=(
            jax.ShapeDtypeStruct((_N, _DW), jnp.uint32),
            jax.ShapeDtypeStruct((_N, _DW), jnp.uint32),
        ),
    )(x1, x2, W)


def _expand_scale(wbuf, fbuf, val_t, boff):
    mask = jnp.full((16,), 0xFFFF0000, jnp.uint32)

    @plsc.parallel_loop(0, _KB, step=1, unroll=2)
    def _(e):
        vv = plsc.load_gather(val_t, [jnp.full((16,), boff, jnp.int32) + e])
        for c in range(_DW // 16):
            w = wbuf[e, pl.ds(c * 16, 16)]
            lo = plsc.bitcast(w << 16, jnp.float32)
            hi = plsc.bitcast(w & mask, jnp.float32)
            fbuf[e, pl.ds(c * 16, 16)] = lo * vv
            fbuf[e, pl.ds(_DW + c * 16, 16)] = hi * vv


def _one_spmm(dst2, src2, val1, fts, out, acc,
              src_t, dst_t, val_t, bufs, tid):
    w0, w1, f0, f1, gs0, gs1, ss0, ss1 = bufs

    def zb(i, c):
        for d in range(8):
            f0[i, pl.ds(d * 16, 16)] = jnp.zeros((16,), jnp.float32)
        return c

    lax.fori_loop(0, _KB, zb, 0, unroll=False)
    zrow = tid * _RPT
    for i in range(_RPT // 40):
        pltpu.sync_copy(f0.at[pl.ds(0, 40)], acc.at[pl.ds(zrow + i * 40, 40)])
    plsc.subcore_barrier()

    n_pairs = _CB // 2

    def chunk(ci, carry):
        base = tid * _NBATCH + ci * _CB
        pltpu.sync_copy(src2.at[pl.ds(base, _CB)], src_t)
        pltpu.sync_copy(dst2.at[pl.ds(base, _CB)], dst_t)
        pltpu.sync_copy(val1.at[pl.ds(tid * _EPT + ci * _CB * _KB, _CB * _KB)],
                        val_t)
        pltpu.async_copy(fts.at[src_t.at[0]], w0, gs0)
        pltpu.async_copy(fts.at[src_t.at[1]], w1, gs1)

        def pair(si, c1):
            b0 = 2 * si
            b1 = b0 + 1
            pltpu.make_async_copy(fts.at[src_t.at[b0]], w0, gs0).wait()

            @pl.when(si > 0)
            def _():
                pltpu.make_async_copy(f0, acc.at[dst_t.at[b0]], ss0).wait()

            _expand_scale(w0, f0, val_t, b0 * _KB)

            @pl.when(b0 + 2 < _CB)
            def _():
                pltpu.async_copy(fts.at[src_t.at[b0 + 2]], w0, gs0)

            pltpu.async_copy(f0, acc.at[dst_t.at[b0]], ss0, add=True)

            pltpu.make_async_copy(fts.at[src_t.at[b1]], w1, gs1).wait()

            @pl.when(si > 0)
            def _():
                pltpu.make_async_copy(f1, acc.at[dst_t.at[b1]], ss1).wait()

            _expand_scale(w1, f1, val_t, b1 * _KB)

            @pl.when(b1 + 2 < _CB)
            def _():
                pltpu.async_copy(fts.at[src_t.at[b1 + 2]], w1, gs1)

            pltpu.async_copy(f1, acc.at[dst_t.at[b1]], ss1, add=True)
            return c1

        lax.fori_loop(0, n_pairs, pair, 0, unroll=False)
        pltpu.make_async_copy(f0, acc.at[dst_t.at[0]], ss0).wait()
        pltpu.make_async_copy(f1, acc.at[dst_t.at[0]], ss1).wait()
        return carry

    lax.fori_loop(0, _NCHUNK, chunk, 0, unroll=False)
    plsc.subcore_barrier()

    orow = tid * _RPT

    @pl.when(tid < _NT - 1)
    def _():
        pltpu.sync_copy(acc.at[pl.ds(orow, _RPT)], out.at[pl.ds(orow, _RPT)])

    @pl.when(tid == _NT - 1)
    def _():
        lo = (_NT - 1) * _RPT
        pltpu.sync_copy(acc.at[pl.ds(lo, _N - lo)], out.at[pl.ds(lo, _N - lo)])

    plsc.subcore_barrier()


def _spmm_body(fts1, fts2,
               dA, sA, vA, d1, s1, v1, d2, s2, v2,
               s0_out, s1_out, s2_out, s3_out,
               acc, src_t, dst_t, val_t,
               w0, w1, f0, f1, gs0, gs1, ss0, ss1):
    cid = lax.axis_index("c")
    tid = lax.axis_index("s")
    bufs = (w0, w1, f0, f1, gs0, gs1, ss0, ss1)

    @pl.when(cid == 0)
    def _():
        _one_spmm(dA, sA, vA, fts1, s0_out, acc, src_t, dst_t, val_t, bufs, tid)
        _one_spmm(d1, s1, v1, fts1, s1_out, acc, src_t, dst_t, val_t, bufs, tid)

    @pl.when(cid == 1)
    def _():
        _one_spmm(d2, s2, v2, fts1, s2_out, acc, src_t, dst_t, val_t, bufs, tid)
        _one_spmm(dA, sA, vA, fts2, s3_out, acc, src_t, dst_t, val_t, bufs, tid)


def _spmm_all(fts1, fts2, dA, sA, vA, d1, s1, v1, d2, s2, v2):
    f32 = jnp.float32
    run = pl.kernel(
        _spmm_body,
        out_type=[jax.ShapeDtypeStruct((_N, _D), f32) for _ in range(4)],
        mesh=plsc.VectorSubcoreMesh(core_axis_name="c", subcore_axis_name="s"),
        compiler_params=pltpu.CompilerParams(needs_layout_passes=False, use_tc_tiling_on_sc=False),
        scratch_types=[
            pltpu.VMEM_SHARED((_ACC_N, _D), f32),
            pltpu.VMEM((_CB, _KB), jnp.int32),
            pltpu.VMEM((_CB, _KB), jnp.int32),
            pltpu.VMEM((_CB * _KB,), f32),
            pltpu.VMEM((_KB, _DW), jnp.uint32),
            pltpu.VMEM((_KB, _DW), jnp.uint32),
            pltpu.VMEM((_KB, _D), f32),
            pltpu.VMEM((_KB, _D), f32),
            pltpu.SemaphoreType.DMA,
            pltpu.SemaphoreType.DMA,
            pltpu.SemaphoreType.DMA,
            pltpu.SemaphoreType.DMA,
        ],
    )
    return run(fts1, fts2, dA, sA, vA, d1, s1, v1, d2, s2, v2)


def _fin_body(s0_ref, s1_ref, s2_ref, s3_ref, b_ref, a_ref, wb_ref, bb_ref,
              out_ref):
    a = a_ref[0, 0]
    b = b_ref[...]

    def prelu(x):
        return jnp.where(x > 0, x, a * x)

    h1 = prelu(s1_ref[...] + b)
    h3 = prelu(s2_ref[...] + b)
    c = jax.nn.sigmoid(jnp.mean(h1, axis=0)) + jax.nn.sigmoid(jnp.mean(h3, axis=0))
    v = jnp.sum(wb_ref[...] * c[None, :], axis=1)

    h0 = prelu(s0_ref[...] + b)
    h2 = prelu(s3_ref[...] + b)
    two_bb = 2.0 * bb_ref[0, 0]
    out_ref[0, :] = jnp.sum(h0 * v[None, :], axis=1) + two_bb
    out_ref[1, :] = jnp.sum(h2 * v[None, :], axis=1) + two_bb


def _finalize(s0, s1, s2, s3, b, a, Wb, bb):
    return pl.pallas_call(
        _fin_body,
        out_shape=jax.ShapeDtypeStruct((2, _N), jnp.float32),
    )(s0, s1, s2, s3, b, a, Wb, bb)


def kernel(seq1, seq2, adj_idx, adj_val, aug1_idx, aug1_val, aug2_idx,
           aug2_val, W, b_gcn, prelu_a, Wb, bb):
    fts1, fts2 = _features(seq1[0], seq2[0], W)

    def layout(idx, val):
        dst = idx[0].reshape(_E // _KB, _KB)
        src = idx[1].reshape(_E // _KB, _KB)
        return dst, src, val

    dA, sA, vA = layout(adj_idx, adj_val)
    d1, s1, v1 = layout(aug1_idx, aug1_val)
    d2, s2, v2 = layout(aug2_idx, aug2_val)

    s0, s1_, s2_, s3 = _spmm_all(fts1, fts2, dA, sA, vA, d1, s1, v1, d2, s2, v2)

    out2 = _finalize(s0, s1_, s2_, s3,
                     b_gcn.reshape(1, _D),
                     prelu_a.reshape(1, 1),
                     Wb,
                     bb.reshape(1, 1))
    return out2.reshape(1, 2 * _N)


# speedup vs baseline: 10.1432x; 1.1083x over previous
import jax
import jax.numpy as jnp
from jax import lax
from jax.experimental import pallas as pl
from jax.experimental.pallas import tpu as pltpu
from jax.experimental.pallas import tpu_sc as plsc

_N = 10000
_E = 320000
_D = 128
_DW = _D // 2
_NT = 16
_KB = 50
_EPT = _E // _NT
_NBATCH = _EPT // _KB
_ACC_N = 10240
_RPT = _ACC_N // _NT
_CB = 40
_NCHUNK = _NBATCH // _CB


def _mm_body(x1_ref, x2_ref, w_ref, o1_ref, o2_ref):
    w = w_ref[...]

    def pack(x):
        o = jnp.dot(x, w, preferred_element_type=jnp.float32)
        u = lax.bitcast_convert_type(o.astype(jnp.bfloat16), jnp.uint16)
        lo = u[:, :_DW].astype(jnp.uint32)
        hi = u[:, _DW:].astype(jnp.uint32)
        return lo | (hi << 16)

    o1_ref[...] = pack(x1_ref[...])
    o2_ref[...] = pack(x2_ref[...])


def _features(x1, x2, W):
    return pl.pallas_call(
        _mm_body,
        out_shape=(
            jax.ShapeDtypeStruct((_N, _DW), jnp.uint32),
            jax.ShapeDtypeStruct((_N, _DW), jnp.uint32),
        ),
    )(x1, x2, W)


def _expand_scale(wbuf, fbuf, val_t, boff):
    mask = jnp.full((16,), 0xFFFF0000, jnp.uint32)

    @plsc.parallel_loop(0, _KB, step=1, unroll=2)
    def _(e):
        vv = plsc.load_gather(val_t, [jnp.full((16,), boff, jnp.int32) + e])
        for c in range(_DW // 16):
            w = wbuf[e, pl.ds(c * 16, 16)]
            lo = plsc.bitcast(w << 16, jnp.float32)
            hi = plsc.bitcast(w & mask, jnp.float32)
            fbuf[e, pl.ds(c * 16, 16)] = lo * vv
            fbuf[e, pl.ds(_DW + c * 16, 16)] = hi * vv


def _one_spmm(dst2, src2, val1, fts, out, acc,
              src_t, dst_t, val_t, bufs, tid):
    w0, w1, w2, w3, f0, f1, gs0, gs1, gs2, gs3, ss0, ss1 = bufs

    def zb(i, c):
        for d in range(8):
            f0[i, pl.ds(d * 16, 16)] = jnp.zeros((16,), jnp.float32)
        return c

    lax.fori_loop(0, _KB, zb, 0, unroll=False)
    zrow = tid * _RPT
    for i in range(_RPT // 40):
        pltpu.sync_copy(f0.at[pl.ds(0, 40)], acc.at[pl.ds(zrow + i * 40, 40)])
    plsc.subcore_barrier()

    def chunk(ci, carry):
        base = tid * _NBATCH + ci * _CB
        pltpu.sync_copy(src2.at[pl.ds(base, _CB)], src_t)
        pltpu.sync_copy(dst2.at[pl.ds(base, _CB)], dst_t)
        pltpu.sync_copy(val1.at[pl.ds(tid * _EPT + ci * _CB * _KB, _CB * _KB)],
                        val_t)
        ws = (w0, w1, w2, w3)
        gss = (gs0, gs1, gs2, gs3)
        fs = (f0, f1)
        sss = (ss0, ss1)
        for j in range(3):
            pltpu.async_copy(fts.at[src_t.at[j]], ws[j], gss[j])

        def quad(g, c1):
            for j in range(4):
                b = 4 * g + j
                wj, gj = ws[j], gss[j]
                fj, sj = fs[j % 2], sss[j % 2]
                pltpu.make_async_copy(fts.at[src_t.at[b]], wj, gj).wait()
                if j < 2:
                    @pl.when(g > 0)
                    def _():
                        pltpu.make_async_copy(fj, acc.at[dst_t.at[b]], sj).wait()
                else:
                    pltpu.make_async_copy(fj, acc.at[dst_t.at[b]], sj).wait()

                _expand_scale(wj, fj, val_t, b * _KB)

                nw = (j + 3) % 4

                @pl.when(b + 3 < _CB)
                def _():
                    pltpu.async_copy(fts.at[src_t.at[b + 3]], ws[nw], gss[nw])

                pltpu.async_copy(fj, acc.at[dst_t.at[b]], sj, add=True)
            return c1

        lax.fori_loop(0, _CB // 4, quad, 0, unroll=False)
        pltpu.make_async_copy(f0, acc.at[dst_t.at[0]], ss0).wait()
        pltpu.make_async_copy(f1, acc.at[dst_t.at[0]], ss1).wait()
        return carry

    lax.fori_loop(0, _NCHUNK, chunk, 0, unroll=False)
    plsc.subcore_barrier()

    orow = tid * _RPT

    @pl.when(tid < _NT - 1)
    def _():
        pltpu.sync_copy(acc.at[pl.ds(orow, _RPT)], out.at[pl.ds(orow, _RPT)])

    @pl.when(tid == _NT - 1)
    def _():
        lo = (_NT - 1) * _RPT
        pltpu.sync_copy(acc.at[pl.ds(lo, _N - lo)], out.at[pl.ds(lo, _N - lo)])

    plsc.subcore_barrier()


def _spmm_body(fts1, fts2,
               dA, sA, vA, d1, s1, v1, d2, s2, v2,
               s0_out, s1_out, s2_out, s3_out,
               acc, src_t, dst_t, val_t,
               w0, w1, w2, w3, f0, f1,
               gs0, gs1, gs2, gs3, ss0, ss1):
    cid = lax.axis_index("c")
    tid = lax.axis_index("s")
    bufs = (w0, w1, w2, w3, f0, f1, gs0, gs1, gs2, gs3, ss0, ss1)

    @pl.when(cid == 0)
    def _():
        _one_spmm(dA, sA, vA, fts1, s0_out, acc, src_t, dst_t, val_t, bufs, tid)
        _one_spmm(d1, s1, v1, fts1, s1_out, acc, src_t, dst_t, val_t, bufs, tid)

    @pl.when(cid == 1)
    def _():
        _one_spmm(d2, s2, v2, fts1, s2_out, acc, src_t, dst_t, val_t, bufs, tid)
        _one_spmm(dA, sA, vA, fts2, s3_out, acc, src_t, dst_t, val_t, bufs, tid)


def _spmm_all(fts1, fts2, dA, sA, vA, d1, s1, v1, d2, s2, v2):
    f32 = jnp.float32
    run = pl.kernel(
        _spmm_body,
        out_type=[jax.ShapeDtypeStruct((_N, _D), f32) for _ in range(4)],
        mesh=plsc.VectorSubcoreMesh(core_axis_name="c", subcore_axis_name="s"),
        compiler_params=pltpu.CompilerParams(needs_layout_passes=False, use_tc_tiling_on_sc=False),
        scratch_types=[
            pltpu.VMEM_SHARED((_ACC_N, _D), f32),
            pltpu.VMEM((_CB, _KB), jnp.int32),
            pltpu.VMEM((_CB, _KB), jnp.int32),
            pltpu.VMEM((_CB * _KB,), f32),
            pltpu.VMEM((_KB, _DW), jnp.uint32),
            pltpu.VMEM((_KB, _DW), jnp.uint32),
            pltpu.VMEM((_KB, _DW), jnp.uint32),
            pltpu.VMEM((_KB, _DW), jnp.uint32),
            pltpu.VMEM((_KB, _D), f32),
            pltpu.VMEM((_KB, _D), f32),
            pltpu.SemaphoreType.DMA,
            pltpu.SemaphoreType.DMA,
            pltpu.SemaphoreType.DMA,
            pltpu.SemaphoreType.DMA,
            pltpu.SemaphoreType.DMA,
            pltpu.SemaphoreType.DMA,
        ],
    )
    return run(fts1, fts2, dA, sA, vA, d1, s1, v1, d2, s2, v2)


def _fin_body(s0_ref, s1_ref, s2_ref, s3_ref, b_ref, a_ref, wb_ref, bb_ref,
              out_ref):
    a = a_ref[0, 0]
    b = b_ref[...]

    def prelu(x):
        return jnp.where(x > 0, x, a * x)

    h1 = prelu(s1_ref[...] + b)
    h3 = prelu(s2_ref[...] + b)
    c = jax.nn.sigmoid(jnp.mean(h1, axis=0)) + jax.nn.sigmoid(jnp.mean(h3, axis=0))
    v = jnp.sum(wb_ref[...] * c[None, :], axis=1)

    h0 = prelu(s0_ref[...] + b)
    h2 = prelu(s3_ref[...] + b)
    two_bb = 2.0 * bb_ref[0, 0]
    out_ref[0, :] = jnp.sum(h0 * v[None, :], axis=1) + two_bb
    out_ref[1, :] = jnp.sum(h2 * v[None, :], axis=1) + two_bb


def _finalize(s0, s1, s2, s3, b, a, Wb, bb):
    return pl.pallas_call(
        _fin_body,
        out_shape=jax.ShapeDtypeStruct((2, _N), jnp.float32),
    )(s0, s1, s2, s3, b, a, Wb, bb)


def kernel(seq1, seq2, adj_idx, adj_val, aug1_idx, aug1_val, aug2_idx,
           aug2_val, W, b_gcn, prelu_a, Wb, bb):
    fts1, fts2 = _features(seq1[0], seq2[0], W)

    def layout(idx, val):
        dst = idx[0].reshape(_E // _KB, _KB)
        src = idx[1].reshape(_E // _KB, _KB)
        return dst, src, val

    dA, sA, vA = layout(adj_idx, adj_val)
    d1, s1, v1 = layout(aug1_idx, aug1_val)
    d2, s2, v2 = layout(aug2_idx, aug2_val)

    s0, s1_, s2_, s3 = _spmm_all(fts1, fts2, dA, sA, vA, d1, s1, v1, d2, s2, v2)

    out2 = _finalize(s0, s1_, s2_, s3,
                     b_gcn.reshape(1, _D),
                     prelu_a.reshape(1, 1),
                     Wb,
                     bb.reshape(1, 1))
    return out2.reshape(1, 2 * _N)
